# Initial kernel scaffold; baseline (speedup 1.0000x reference)
import jax
import jax.numpy as jnp
from jax.experimental import pallas as pl


def kernel(h, edges, Win, bin_, We1, be1, We2, be2, Wn1, bn1, Wn2, bn2, Wout, bout):
    raise NotImplementedError("write your pallas kernel here")


# speedup vs baseline: 3.2968x; 3.2968x over previous
import functools

import jax
import jax.numpy as jnp
from jax import lax
from jax.experimental import pallas as pl
from jax.experimental.pallas import tpu as pltpu
from jax.experimental.pallas import tpu_sc as plsc

N_LAYERS = 4
C = 1.0
N, E, D, H = 10000, 320000, 128, 128

NC, NS = 2, 16
NW = NC * NS
EC = 128
NCHUNK = E // EC
VPL = H // 16

NP = 10240
ROWS_PER_TILE = NP // NS
ZR = 128

_mesh = plsc.VectorSubcoreMesh(core_axis_name="c", subcore_axis_name="s")


def _wid():
    return lax.axis_index("s") * NC + lax.axis_index("c")


@functools.partial(
    pl.kernel,
    out_type=jax.ShapeDtypeStruct((E, H), jnp.float32),
    mesh=_mesh,
    scratch_types=[
        pltpu.VMEM((EC,), jnp.int32),
        pltpu.VMEM((EC,), jnp.int32),
        pltpu.VMEM((EC, H), jnp.float32),
        pltpu.VMEM((EC, H), jnp.float32),
        pltpu.SemaphoreType.DMA,
        pltpu.SemaphoreType.DMA,
    ],
)
def _sc_gather_add(xa_hbm, xbp_hbm, row_hbm, col_hbm, g_hbm,
                   idxr, idxc, bufa, bufb, sema, semb):
    wid = _wid()
    nloop = (NCHUNK + NW - 1) // NW

    def step(t, _):
        j = wid + t * NW

        @pl.when(j < NCHUNK)
        def _():
            pltpu.sync_copy(row_hbm.at[j], idxr)
            pltpu.sync_copy(col_hbm.at[j], idxc)
            cpa = pltpu.async_copy(xa_hbm.at[idxr], bufa, sema)
            cpb = pltpu.async_copy(xbp_hbm.at[idxc], bufb, semb)
            cpa.wait()
            cpb.wait()

            def add_row(r, _):
                for cidx in range(VPL):
                    sl = pl.ds(cidx * 16, 16)
                    bufa[r, sl] = bufa[r, sl] + bufb[r, sl]
                return 0

            lax.fori_loop(0, EC, add_row, 0)
            pltpu.sync_copy(bufa, g_hbm.at[pl.ds(j * EC, EC)])

        return 0

    lax.fori_loop(0, nloop, step, 0)


@functools.partial(
    pl.kernel,
    out_type=jax.ShapeDtypeStruct((NC, NP, H), jnp.float32),
    mesh=_mesh,
    scratch_types=[
        pltpu.VMEM((EC,), jnp.int32),
        pltpu.VMEM((EC, H), jnp.float32),
        pltpu.VMEM((ZR, H), jnp.float32),
        pltpu.VMEM_SHARED((NP, H), jnp.float32),
    ],
)
def _sc_scatter_add(ef_hbm, row_hbm, aggp_hbm, idx, buf, zbuf, agg_sh):
    cid = lax.axis_index("c")
    sid = lax.axis_index("s")
    wid = _wid()

    def zero_row(r, _):
        for cidx in range(VPL):
            zbuf[r, pl.ds(cidx * 16, 16)] = jnp.zeros((16,), jnp.float32)
        return 0

    lax.fori_loop(0, ZR, zero_row, 0)

    def zero_copy(q, _):
        pltpu.sync_copy(zbuf, agg_sh.at[pl.ds(sid * ROWS_PER_TILE + q * ZR, ZR)])
        return 0

    lax.fori_loop(0, ROWS_PER_TILE // ZR, zero_copy, 0)
    plsc.subcore_barrier()

    nloop = (NCHUNK + NW - 1) // NW

    def step(t, _):
        j = wid + t * NW

        @pl.when(j < NCHUNK)
        def _():
            pltpu.sync_copy(row_hbm.at[j], idx)
            pltpu.sync_copy(ef_hbm.at[pl.ds(j * EC, EC)], buf)
            pltpu.sync_copy(buf, agg_sh.at[idx], add=True)

        return 0

    lax.fori_loop(0, nloop, step, 0)
    plsc.subcore_barrier()

    pltpu.sync_copy(agg_sh.at[pl.ds(sid * ROWS_PER_TILE, ROWS_PER_TILE)],
                    aggp_hbm.at[cid, pl.ds(sid * ROWS_PER_TILE, ROWS_PER_TILE)])


def _silu(x):
    return x * jax.nn.sigmoid(x)


def _tc_input_body(h_ref, win, binr, wea, web, ben, xo, xao, xbo):
    x = jnp.dot(h_ref[...], win[...], preferred_element_type=jnp.float32)
    x = x + binr[...]
    xo[...] = x
    xao[...] = jnp.dot(x, wea[...], preferred_element_type=jnp.float32)
    xbo[...] = jnp.dot(x, web[...], preferred_element_type=jnp.float32) + ben[...]


def _tc_edge_body(g_ref, w2, b2, ef_ref):
    t = _silu(g_ref[...])
    u = jnp.dot(t, w2[...], preferred_element_type=jnp.float32) + b2[...]
    ef_ref[...] = _silu(u)


def _tc_node_body(x_ref, aggp_ref, wn1a, wn1b, bn1r, wn2, bn2r,
                  wea, web, ben, xo, xao, xbo):
    x = x_ref[...]
    agg = (aggp_ref[0] + aggp_ref[1]) * (1.0 / C)
    t = _silu(jnp.dot(x, wn1a[...], preferred_element_type=jnp.float32)
              + jnp.dot(agg, wn1b[...], preferred_element_type=jnp.float32)
              + bn1r[...])
    xn = x + jnp.dot(t, wn2[...], preferred_element_type=jnp.float32) + bn2r[...]
    xo[...] = xn
    xao[...] = jnp.dot(xn, wea[...], preferred_element_type=jnp.float32)
    xbo[...] = jnp.dot(xn, web[...], preferred_element_type=jnp.float32) + ben[...]


def _tc_node_final_body(x_ref, aggp_ref, wn1a, wn1b, bn1r, wn2, bn2r,
                        wout, boutr, yo):
    x = x_ref[...]
    agg = (aggp_ref[0] + aggp_ref[1]) * (1.0 / C)
    t = _silu(jnp.dot(x, wn1a[...], preferred_element_type=jnp.float32)
              + jnp.dot(agg, wn1b[...], preferred_element_type=jnp.float32)
              + bn1r[...])
    xn = x + jnp.dot(t, wn2[...], preferred_element_type=jnp.float32) + bn2r[...]
    yo[...] = jnp.dot(xn, wout[...], preferred_element_type=jnp.float32) + boutr[...]


BN = 2000
BE = 3200


def _wspec(shape):
    return pl.BlockSpec(shape, lambda i: (0,) * len(shape))


_node_out = [jax.ShapeDtypeStruct((N, H), jnp.float32)] * 3
_nblock = pl.BlockSpec((BN, H), lambda i: (i, 0))
_ablock = pl.BlockSpec((NC, BN, H), lambda i: (0, i, 0))

_tc_input = pl.pallas_call(
    _tc_input_body,
    grid=(N // BN,),
    in_specs=[_nblock, _wspec((D, H)), _wspec((1, H)), _wspec((H, H)),
              _wspec((H, H)), _wspec((1, H))],
    out_specs=[_nblock] * 3,
    out_shape=_node_out,
)

_tc_edge = pl.pallas_call(
    _tc_edge_body,
    grid=(E // BE,),
    in_specs=[pl.BlockSpec((BE, H), lambda i: (i, 0)), _wspec((H, H)),
              _wspec((1, H))],
    out_specs=pl.BlockSpec((BE, H), lambda i: (i, 0)),
    out_shape=jax.ShapeDtypeStruct((E, H), jnp.float32),
)

_tc_node = pl.pallas_call(
    _tc_node_body,
    grid=(N // BN,),
    in_specs=[_nblock, _ablock] + [_wspec((H, H)), _wspec((H, H)),
              _wspec((1, H)), _wspec((H, H)), _wspec((1, H)),
              _wspec((H, H)), _wspec((H, H)), _wspec((1, H))],
    out_specs=[_nblock] * 3,
    out_shape=_node_out,
)

_tc_node_final = pl.pallas_call(
    _tc_node_final_body,
    grid=(N // BN,),
    in_specs=[_nblock, _ablock] + [_wspec((H, H)), _wspec((H, H)),
              _wspec((1, H)), _wspec((H, H)), _wspec((1, H)),
              _wspec((H, D)), _wspec((1, D))],
    out_specs=pl.BlockSpec((BN, D), lambda i: (i, 0)),
    out_shape=jax.ShapeDtypeStruct((N, D), jnp.float32),
)


def kernel(h, edges, Win, bin_, We1, be1, We2, be2, Wn1, bn1, Wn2, bn2,
           Wout, bout):
    row2d = edges[0].reshape(NCHUNK, EC)
    col2d = edges[1].reshape(NCHUNK, EC)
    b2 = lambda v: v.reshape(1, -1)

    x, xa, xbp = _tc_input(h, Win, b2(bin_), We1[0, :H], We1[0, H:], b2(be1[0]))
    for i in range(N_LAYERS):
        g = _sc_gather_add(xa, xbp, row2d, col2d)
        ef = _tc_edge(g, We2[i], b2(be2[i]))
        aggp = _sc_scatter_add(ef, row2d)
        if i < N_LAYERS - 1:
            x, xa, xbp = _tc_node(x, aggp, Wn1[i, :H], Wn1[i, H:], b2(bn1[i]),
                                  Wn2[i], b2(bn2[i]), We1[i + 1, :H],
                                  We1[i + 1, H:], b2(be1[i + 1]))
        else:
            y = _tc_node_final(x, aggp, Wn1[i, :H], Wn1[i, H:], b2(bn1[i]),
                               Wn2[i], b2(bn2[i]), Wout, b2(bout))
    return y
